# initial kernel scaffold (unmeasured)
import jax
import jax.numpy as jnp
from jax import lax
from jax.experimental import pallas as pl
from jax.experimental.pallas import tpu as pltpu

N_DEV = 32


def kernel(x, w_mat):
    m_per, k = x.shape
    _, n_per = w_mat.shape
    m_glob = N_DEV * m_per

    def body(x_ref, w_ref, out_ref, gx_ref, xbf_ref, send_sems, recv_sems):
        my = lax.axis_index("i")

        xbf_ref[...] = x_ref[...].astype(jnp.bfloat16)
        gx_ref[pl.ds(my * m_per, m_per), :] = xbf_ref[...]

        sends = []
        for off in range(1, N_DEV):
            peer = lax.rem(my + off, N_DEV)
            rdma = pltpu.make_async_remote_copy(
                src_ref=xbf_ref,
                dst_ref=gx_ref.at[pl.ds(my * m_per, m_per), :],
                send_sem=send_sems.at[off],
                recv_sem=recv_sems.at[off],
                device_id=(peer,),
                device_id_type=pl.DeviceIdType.MESH,
            )
            rdma.start()
            sends.append(rdma)

        for off in range(1, N_DEV):
            src_pos = lax.rem(my + N_DEV - off, N_DEV)
            recv = pltpu.make_async_remote_copy(
                src_ref=xbf_ref,
                dst_ref=gx_ref.at[pl.ds(src_pos * m_per, m_per), :],
                send_sem=send_sems.at[off],
                recv_sem=recv_sems.at[off],
                device_id=(src_pos,),
                device_id_type=pl.DeviceIdType.MESH,
            )
            recv.wait_recv()

        wbf = w_ref[...].astype(jnp.bfloat16)
        y = jnp.dot(gx_ref[...], wbf, preferred_element_type=jnp.float32)
        c = 0.7978845608028654
        out_ref[...] = 0.5 * y * (1.0 + jnp.tanh(c * (y + 0.044715 * y * y * y)))

        for rdma in sends:
            rdma.wait_send()

    return pl.pallas_call(
        body,
        out_shape=jax.ShapeDtypeStruct((m_glob, n_per), jnp.float32),
        in_specs=[
            pl.BlockSpec(memory_space=pltpu.VMEM),
            pl.BlockSpec(memory_space=pltpu.VMEM),
        ],
        out_specs=pl.BlockSpec(memory_space=pltpu.VMEM),
        scratch_shapes=[
            pltpu.VMEM((m_glob, k), jnp.bfloat16),
            pltpu.VMEM((m_per, k), jnp.bfloat16),
            pltpu.SemaphoreType.DMA((N_DEV,)),
            pltpu.SemaphoreType.DMA((N_DEV,)),
        ],
        compiler_params=pltpu.CompilerParams(collective_id=0),
    )(x, w_mat)


# baseline (device time: 123420 ns/iter reference)
import jax
import jax.numpy as jnp
from jax import lax
from jax.experimental import pallas as pl
from jax.experimental.pallas import tpu as pltpu

N_DEV = 32


def kernel(x, w_mat):
    m_per, k = x.shape
    _, n_per = w_mat.shape
    m_glob = N_DEV * m_per

    def body(x_ref, w_ref, out_ref, gx_ref, xbf_ref, send_sems, recv_sems):
        my = lax.axis_index("i")

        xbf_ref[...] = x_ref[...].astype(jnp.bfloat16)
        gx_ref[pl.ds(my * m_per, m_per), :] = xbf_ref[...]

        sends = []
        for off in range(1, N_DEV):
            peer = lax.rem(my + off, N_DEV)
            rdma = pltpu.make_async_remote_copy(
                src_ref=xbf_ref,
                dst_ref=gx_ref.at[pl.ds(my * m_per, m_per), :],
                send_sem=send_sems.at[off],
                recv_sem=recv_sems.at[off],
                device_id=(peer,),
                device_id_type=pl.DeviceIdType.MESH,
            )
            rdma.start()
            sends.append(rdma)

        for off in range(1, N_DEV):
            src_pos = lax.rem(my + N_DEV - off, N_DEV)
            recv = pltpu.make_async_remote_copy(
                src_ref=xbf_ref,
                dst_ref=gx_ref.at[pl.ds(src_pos * m_per, m_per), :],
                send_sem=send_sems.at[off],
                recv_sem=recv_sems.at[off],
                device_id=(src_pos,),
                device_id_type=pl.DeviceIdType.MESH,
            )
            recv.wait_recv()

        wbf = w_ref[...].astype(jnp.bfloat16)
        y = jnp.dot(gx_ref[...], wbf, preferred_element_type=jnp.float32)
        c = 0.7978845608028654
        out_ref[...] = 0.5 * y * (1.0 + jnp.tanh(c * (y + 0.044715 * y * y * y)))

        for rdma in sends:
            rdma.wait_send()

    return pl.pallas_call(
        body,
        out_shape=jax.ShapeDtypeStruct((m_glob, n_per), jnp.float32),
        in_specs=[
            pl.BlockSpec(memory_space=pltpu.VMEM),
            pl.BlockSpec(memory_space=pltpu.VMEM),
        ],
        out_specs=pl.BlockSpec(memory_space=pltpu.VMEM),
        scratch_shapes=[
            pltpu.VMEM((m_glob, k), jnp.bfloat16),
            pltpu.VMEM((m_per, k), jnp.bfloat16),
            pltpu.SemaphoreType.DMA((N_DEV,)),
            pltpu.SemaphoreType.DMA((N_DEV,)),
        ],
    )(x, w_mat)


# device time: 60580 ns/iter; 2.0373x vs baseline; 2.0373x over previous
import numpy as np

import jax
import jax.numpy as jnp
from jax import lax
from jax.experimental import pallas as pl
from jax.experimental.pallas import tpu as pltpu

N_DEV = 32
M_PER = 64
SUB = 4
SUB_ROWS = M_PER // SUB
NF = N_DEV // 2
NB = N_DEV // 2 - 1

_PLANE = [(0, 0), (1, 0), (1, 1), (0, 1), (0, 2), (1, 2), (1, 3), (0, 3)]


def _lid_to_coords(p):
    z, r = divmod(p, 8)
    x, y = _PLANE[r]
    return (x, y, z)


_COORDS_TO_LID = {_lid_to_coords(p): p for p in range(N_DEV)}

_SEQ0 = [
    (0, 0), (1, 0), (2, 0), (3, 0),
    (3, 1), (2, 1), (1, 1), (0, 1),
    (0, 2), (1, 2), (2, 2), (3, 2),
    (3, 3), (2, 3), (1, 3), (0, 3),
]
_CYCLE = [(0, y, z) for (y, z) in _SEQ0] + [(1, y, z) for (y, z) in reversed(_SEQ0)]
for _a, _b in zip(_CYCLE, _CYCLE[1:] + _CYCLE[:1]):
    assert sum(abs(u - v) for u, v in zip(_a, _b)) == 1, (_a, _b)

_PERM = [_COORDS_TO_LID[c] for c in _CYCLE]
_CPOS = {lid: i for i, lid in enumerate(_PERM)}

_TBL = np.zeros((N_DEV, 2 + NF + NF + NB + NB), dtype=np.int32)
for _d in range(N_DEV):
    cp = _CPOS[_d]
    row = [_PERM[(cp + 1) % N_DEV], _PERM[(cp - 1) % N_DEV]]
    row += [_PERM[(cp - h) % N_DEV] * M_PER for h in range(NF)]
    row += [_PERM[(cp - 1 - h) % N_DEV] * M_PER for h in range(NF)]
    row += [_PERM[(cp + h) % N_DEV] * M_PER for h in range(NB)]
    row += [_PERM[(cp + 1 + h) % N_DEV] * M_PER for h in range(NB)]
    _TBL[_d] = row
_OFF_SF, _OFF_OF, _OFF_SB, _OFF_OB = 2, 2 + NF, 2 + 2 * NF, 2 + 2 * NF + NB


def kernel(x, w_mat):
    m_per, k = x.shape
    _, n_per = w_mat.shape
    m_glob = N_DEV * m_per

    my = lax.axis_index("i")
    meta = jnp.take(jnp.asarray(_TBL), my, axis=0)

    def body(x_ref, w_ref, meta_ref, out_ref, gx_ref,
             fs_sems, fr_sems, bs_sems, br_sems):
        fwd_tgt = meta_ref[0]
        bwd_tgt = meta_ref[1]

        my_row = pl.multiple_of(meta_ref[_OFF_SF], m_per)
        gx_ref[pl.ds(my_row, m_per), :] = x_ref[...].astype(jnp.bfloat16)

        barrier = pltpu.get_barrier_semaphore()
        for tgt in (fwd_tgt, bwd_tgt):
            pl.semaphore_signal(
                barrier, inc=1,
                device_id=(tgt,), device_id_type=pl.DeviceIdType.MESH,
            )
        pl.semaphore_wait(barrier, 2)

        def rdma(row, tgt, send_sem, recv_sem):
            row = pl.multiple_of(row, SUB_ROWS)
            return pltpu.make_async_remote_copy(
                src_ref=gx_ref.at[pl.ds(row, SUB_ROWS), :],
                dst_ref=gx_ref.at[pl.ds(row, SUB_ROWS), :],
                send_sem=send_sem,
                recv_sem=recv_sem,
                device_id=(tgt,),
                device_id_type=pl.DeviceIdType.MESH,
            )

        def f_send(h, s):
            return rdma(meta_ref[_OFF_SF + h] + s * SUB_ROWS, fwd_tgt,
                        fs_sems.at[h, s], fr_sems.at[h, s])

        def f_recv(h, s):
            return rdma(meta_ref[_OFF_OF + h] + s * SUB_ROWS, fwd_tgt,
                        fs_sems.at[h, s], fr_sems.at[h, s])

        def b_send(h, s):
            return rdma(meta_ref[_OFF_SB + h] + s * SUB_ROWS, bwd_tgt,
                        bs_sems.at[h, s], br_sems.at[h, s])

        def b_recv(h, s):
            return rdma(meta_ref[_OFF_OB + h] + s * SUB_ROWS, bwd_tgt,
                        bs_sems.at[h, s], br_sems.at[h, s])

        sends = []
        for s in range(SUB):
            d = f_send(0, s); d.start(); sends.append(d)
            d = b_send(0, s); d.start(); sends.append(d)

        for h in range(1, NF):
            for s in range(SUB):
                f_recv(h - 1, s).wait_recv()
                d = f_send(h, s); d.start(); sends.append(d)
            if h < NB:
                for s in range(SUB):
                    b_recv(h - 1, s).wait_recv()
                    d = b_send(h, s); d.start(); sends.append(d)

        for s in range(SUB):
            f_recv(NF - 1, s).wait_recv()
        for s in range(SUB):
            b_recv(NB - 1, s).wait_recv()

        wbf = w_ref[...].astype(jnp.bfloat16)
        y = jnp.dot(gx_ref[...], wbf, preferred_element_type=jnp.float32)
        c = 0.7978845608028654
        out_ref[...] = 0.5 * y * (1.0 + jnp.tanh(c * (y + 0.044715 * y * y * y)))

        for d in sends:
            d.wait_send()

    return pl.pallas_call(
        body,
        out_shape=jax.ShapeDtypeStruct((m_glob, n_per), jnp.float32),
        in_specs=[
            pl.BlockSpec(memory_space=pltpu.VMEM),
            pl.BlockSpec(memory_space=pltpu.VMEM),
            pl.BlockSpec(memory_space=pltpu.SMEM),
        ],
        out_specs=pl.BlockSpec(memory_space=pltpu.VMEM),
        scratch_shapes=[
            pltpu.VMEM((m_glob, k), jnp.bfloat16),
            pltpu.SemaphoreType.DMA((NF, SUB)),
            pltpu.SemaphoreType.DMA((NF, SUB)),
            pltpu.SemaphoreType.DMA((NB, SUB)),
            pltpu.SemaphoreType.DMA((NB, SUB)),
        ],
        compiler_params=pltpu.CompilerParams(collective_id=0),
    )(x, w_mat, meta)


# device time: 60346 ns/iter; 2.0452x vs baseline; 1.0039x over previous
import numpy as np

import jax
import jax.numpy as jnp
from jax import lax
from jax.experimental import pallas as pl
from jax.experimental.pallas import tpu as pltpu

N_DEV = 32
M_PER = 64
SUB = 4
SUB_ROWS = M_PER // SUB
NO = 8
NA = 7

_PLANE = [(0, 0), (1, 0), (1, 1), (0, 1), (0, 2), (1, 2), (1, 3), (0, 3)]


def _lid_to_coords(p):
    z, r = divmod(p, 8)
    x, y = _PLANE[r]
    return (x, y, z)


_COORDS_TO_LID = {_lid_to_coords(p): p for p in range(N_DEV)}

_SEQ0 = [
    (0, 0), (1, 0), (2, 0), (3, 0),
    (3, 1), (2, 1), (1, 1), (0, 1),
    (0, 2), (1, 2), (2, 2), (3, 2),
    (3, 3), (2, 3), (1, 3), (0, 3),
]
_CYCLE = [(0, y, z) for (y, z) in _SEQ0] + [(1, y, z) for (y, z) in reversed(_SEQ0)]
for _a, _b in zip(_CYCLE, _CYCLE[1:] + _CYCLE[:1]):
    assert sum(abs(u - v) for u, v in zip(_a, _b)) == 1, (_a, _b)

_PERM = [_COORDS_TO_LID[c] for c in _CYCLE]
_CPOS = {lid: i for i, lid in enumerate(_PERM)}

_TBL = np.zeros((N_DEV, 3 + (NO + 1) + (NA + 1) + (NO + 1) + (NA + 1)),
                dtype=np.int32)
for _d in range(N_DEV):
    cp = _CPOS[_d]
    row = [_PERM[(cp + 1) % N_DEV], _PERM[(cp - 1) % N_DEV],
           _PERM[(cp + 16) % N_DEV]]
    row += [_PERM[(cp - j) % N_DEV] * M_PER for j in range(NO + 1)]
    row += [_PERM[(cp + 16 - j) % N_DEV] * M_PER for j in range(NA + 1)]
    row += [_PERM[(cp + j) % N_DEV] * M_PER for j in range(NO + 1)]
    row += [_PERM[(cp + 16 + j) % N_DEV] * M_PER for j in range(NA + 1)]
    _TBL[_d] = row
_FO = 3
_FA = _FO + NO + 1
_BO = _FA + NA + 1
_BA = _BO + NO + 1

for _d in range(N_DEV):
    got = {_TBL[_d][_FO]}
    for j in range(NO):
        got.add(_TBL[_d][_FO + 1 + j]); got.add(_TBL[_d][_BO + 1 + j])
    for j in range(NA):
        got.add(_TBL[_d][_FA + 1 + j]); got.add(_TBL[_d][_BA + 1 + j])
    got.add(_TBL[_d][_FA])
    assert got == {p * M_PER for p in range(N_DEV)}, _d


def kernel(x, w_mat):
    m_per, k = x.shape
    _, n_per = w_mat.shape
    m_glob = N_DEV * m_per

    my = lax.axis_index("i")
    meta = jnp.take(jnp.asarray(_TBL), my, axis=0)

    def body(x_ref, w_ref, meta_ref, out_ref, gx_ref,
             inj_s, inj_r, fo_s, fo_r, fa_s, fa_r,
             bo_s, bo_r, ba_s, ba_r):
        fwd_tgt = meta_ref[0]
        bwd_tgt = meta_ref[1]
        anti_tgt = meta_ref[2]

        my_row = pl.multiple_of(meta_ref[_FO], m_per)
        gx_ref[pl.ds(my_row, m_per), :] = x_ref[...].astype(jnp.bfloat16)

        barrier = pltpu.get_barrier_semaphore()
        for tgt in (fwd_tgt, bwd_tgt, anti_tgt):
            pl.semaphore_signal(
                barrier, inc=1,
                device_id=(tgt,), device_id_type=pl.DeviceIdType.MESH,
            )
        pl.semaphore_wait(barrier, 3)

        def rdma(row, s, tgt, send_sem, recv_sem):
            row = pl.multiple_of(row + s * SUB_ROWS, SUB_ROWS)
            return pltpu.make_async_remote_copy(
                src_ref=gx_ref.at[pl.ds(row, SUB_ROWS), :],
                dst_ref=gx_ref.at[pl.ds(row, SUB_ROWS), :],
                send_sem=send_sem,
                recv_sem=recv_sem,
                device_id=(tgt,),
                device_id_type=pl.DeviceIdType.MESH,
            )

        sends = []

        def start(d):
            d.start()
            sends.append(d)

        def s_send(idx, j, s, tgt, ss, rs):
            start(rdma(meta_ref[idx + j], s, tgt, ss.at[j, s], rs.at[j, s]))

        def s_recv(idx, j, s, tgt, ss, rs):
            rdma(meta_ref[idx + j + 1], s, tgt, ss.at[j, s], rs.at[j, s]).wait_recv()

        for s in range(SUB):
            start(rdma(meta_ref[_FO], s, anti_tgt, inj_s.at[s], inj_r.at[s]))
            s_send(_FO, 0, s, fwd_tgt, fo_s, fo_r)
            s_send(_BO, 0, s, bwd_tgt, bo_s, bo_r)

        for s in range(SUB):
            s_recv(_FO, 0, s, fwd_tgt, fo_s, fo_r)
            s_send(_FO, 1, s, fwd_tgt, fo_s, fo_r)
        for s in range(SUB):
            s_recv(_BO, 0, s, bwd_tgt, bo_s, bo_r)
            s_send(_BO, 1, s, bwd_tgt, bo_s, bo_r)
        for s in range(SUB):
            rdma(meta_ref[_FA], s, anti_tgt, inj_s.at[s], inj_r.at[s]).wait_recv()
            s_send(_FA, 0, s, fwd_tgt, fa_s, fa_r)
            s_send(_BA, 0, s, bwd_tgt, ba_s, ba_r)

        for r in range(2, NO):
            for s in range(SUB):
                s_recv(_FO, r - 1, s, fwd_tgt, fo_s, fo_r)
                s_send(_FO, r, s, fwd_tgt, fo_s, fo_r)
            for s in range(SUB):
                s_recv(_BO, r - 1, s, bwd_tgt, bo_s, bo_r)
                s_send(_BO, r, s, bwd_tgt, bo_s, bo_r)
            for s in range(SUB):
                s_recv(_FA, r - 2, s, fwd_tgt, fa_s, fa_r)
                s_send(_FA, r - 1, s, fwd_tgt, fa_s, fa_r)
            for s in range(SUB):
                s_recv(_BA, r - 2, s, bwd_tgt, ba_s, ba_r)
                s_send(_BA, r - 1, s, bwd_tgt, ba_s, ba_r)

        for s in range(SUB):
            s_recv(_FO, NO - 1, s, fwd_tgt, fo_s, fo_r)
        for s in range(SUB):
            s_recv(_BO, NO - 1, s, bwd_tgt, bo_s, bo_r)
        for s in range(SUB):
            s_recv(_FA, NA - 1, s, fwd_tgt, fa_s, fa_r)
        for s in range(SUB):
            s_recv(_BA, NA - 1, s, bwd_tgt, ba_s, ba_r)

        wbf = w_ref[...].astype(jnp.bfloat16)
        y = jnp.dot(gx_ref[...], wbf, preferred_element_type=jnp.float32)
        c = 0.7978845608028654
        out_ref[...] = 0.5 * y * (1.0 + jnp.tanh(c * (y + 0.044715 * y * y * y)))

        for d in sends:
            d.wait_send()

    return pl.pallas_call(
        body,
        out_shape=jax.ShapeDtypeStruct((m_glob, n_per), jnp.float32),
        in_specs=[
            pl.BlockSpec(memory_space=pltpu.VMEM),
            pl.BlockSpec(memory_space=pltpu.VMEM),
            pl.BlockSpec(memory_space=pltpu.SMEM),
        ],
        out_specs=pl.BlockSpec(memory_space=pltpu.VMEM),
        scratch_shapes=[
            pltpu.VMEM((m_glob, k), jnp.bfloat16),
            pltpu.SemaphoreType.DMA((SUB,)),
            pltpu.SemaphoreType.DMA((SUB,)),
            pltpu.SemaphoreType.DMA((NO, SUB)),
            pltpu.SemaphoreType.DMA((NO, SUB)),
            pltpu.SemaphoreType.DMA((NA, SUB)),
            pltpu.SemaphoreType.DMA((NA, SUB)),
            pltpu.SemaphoreType.DMA((NO, SUB)),
            pltpu.SemaphoreType.DMA((NO, SUB)),
            pltpu.SemaphoreType.DMA((NA, SUB)),
            pltpu.SemaphoreType.DMA((NA, SUB)),
        ],
        compiler_params=pltpu.CompilerParams(collective_id=0),
    )(x, w_mat, meta)


# device time: 59037 ns/iter; 2.0906x vs baseline; 1.0222x over previous
import numpy as np

import jax
import jax.numpy as jnp
from jax import lax
from jax.experimental import pallas as pl
from jax.experimental.pallas import tpu as pltpu

N_DEV = 32
M_PER = 64
SUB = 1
SUB_ROWS = M_PER // SUB
NO = 8
NA = 7

_PLANE = [(0, 0), (1, 0), (1, 1), (0, 1), (0, 2), (1, 2), (1, 3), (0, 3)]


def _lid_to_coords(p):
    z, r = divmod(p, 8)
    x, y = _PLANE[r]
    return (x, y, z)


_COORDS_TO_LID = {_lid_to_coords(p): p for p in range(N_DEV)}

_SEQ0 = [
    (0, 0), (1, 0), (2, 0), (3, 0),
    (3, 1), (2, 1), (1, 1), (0, 1),
    (0, 2), (1, 2), (2, 2), (3, 2),
    (3, 3), (2, 3), (1, 3), (0, 3),
]
_CYCLE = [(0, y, z) for (y, z) in _SEQ0] + [(1, y, z) for (y, z) in reversed(_SEQ0)]
for _a, _b in zip(_CYCLE, _CYCLE[1:] + _CYCLE[:1]):
    assert sum(abs(u - v) for u, v in zip(_a, _b)) == 1, (_a, _b)

_PERM = [_COORDS_TO_LID[c] for c in _CYCLE]
_CPOS = {lid: i for i, lid in enumerate(_PERM)}

_TBL = np.zeros((N_DEV, 3 + (NO + 1) + (NA + 1) + (NO + 1) + (NA + 1)),
                dtype=np.int32)
for _d in range(N_DEV):
    cp = _CPOS[_d]
    row = [_PERM[(cp + 1) % N_DEV], _PERM[(cp - 1) % N_DEV],
           _PERM[(cp + 16) % N_DEV]]
    row += [_PERM[(cp - j) % N_DEV] * M_PER for j in range(NO + 1)]
    row += [_PERM[(cp + 16 - j) % N_DEV] * M_PER for j in range(NA + 1)]
    row += [_PERM[(cp + j) % N_DEV] * M_PER for j in range(NO + 1)]
    row += [_PERM[(cp + 16 + j) % N_DEV] * M_PER for j in range(NA + 1)]
    _TBL[_d] = row
_FO = 3
_FA = _FO + NO + 1
_BO = _FA + NA + 1
_BA = _BO + NO + 1

for _d in range(N_DEV):
    got = {_TBL[_d][_FO]}
    for j in range(NO):
        got.add(_TBL[_d][_FO + 1 + j]); got.add(_TBL[_d][_BO + 1 + j])
    for j in range(NA):
        got.add(_TBL[_d][_FA + 1 + j]); got.add(_TBL[_d][_BA + 1 + j])
    got.add(_TBL[_d][_FA])
    assert got == {p * M_PER for p in range(N_DEV)}, _d


def kernel(x, w_mat):
    m_per, k = x.shape
    _, n_per = w_mat.shape
    m_glob = N_DEV * m_per

    my = lax.axis_index("i")
    meta = jnp.take(jnp.asarray(_TBL), my, axis=0)

    def body(x_ref, w_ref, meta_ref, out_ref, gx_ref,
             inj_s, inj_r, fo_s, fo_r, fa_s, fa_r,
             bo_s, bo_r, ba_s, ba_r):
        fwd_tgt = meta_ref[0]
        bwd_tgt = meta_ref[1]
        anti_tgt = meta_ref[2]

        my_row = pl.multiple_of(meta_ref[_FO], m_per)
        gx_ref[pl.ds(my_row, m_per), :] = x_ref[...].astype(jnp.bfloat16)

        barrier = pltpu.get_barrier_semaphore()
        for tgt in (fwd_tgt, bwd_tgt, anti_tgt):
            pl.semaphore_signal(
                barrier, inc=1,
                device_id=(tgt,), device_id_type=pl.DeviceIdType.MESH,
            )
        pl.semaphore_wait(barrier, 3)

        def rdma(row, s, tgt, send_sem, recv_sem):
            row = pl.multiple_of(row + s * SUB_ROWS, SUB_ROWS)
            return pltpu.make_async_remote_copy(
                src_ref=gx_ref.at[pl.ds(row, SUB_ROWS), :],
                dst_ref=gx_ref.at[pl.ds(row, SUB_ROWS), :],
                send_sem=send_sem,
                recv_sem=recv_sem,
                device_id=(tgt,),
                device_id_type=pl.DeviceIdType.MESH,
            )

        sends = []

        def start(d):
            d.start()
            sends.append(d)

        def s_send(idx, j, s, tgt, ss, rs):
            start(rdma(meta_ref[idx + j], s, tgt, ss.at[j, s], rs.at[j, s]))

        def s_recv(idx, j, s, tgt, ss, rs):
            rdma(meta_ref[idx + j + 1], s, tgt, ss.at[j, s], rs.at[j, s]).wait_recv()

        for s in range(SUB):
            start(rdma(meta_ref[_FO], s, anti_tgt, inj_s.at[s], inj_r.at[s]))
            s_send(_FO, 0, s, fwd_tgt, fo_s, fo_r)
            s_send(_BO, 0, s, bwd_tgt, bo_s, bo_r)

        for s in range(SUB):
            s_recv(_FO, 0, s, fwd_tgt, fo_s, fo_r)
            s_send(_FO, 1, s, fwd_tgt, fo_s, fo_r)
        for s in range(SUB):
            s_recv(_BO, 0, s, bwd_tgt, bo_s, bo_r)
            s_send(_BO, 1, s, bwd_tgt, bo_s, bo_r)
        for s in range(SUB):
            rdma(meta_ref[_FA], s, anti_tgt, inj_s.at[s], inj_r.at[s]).wait_recv()
            s_send(_FA, 0, s, fwd_tgt, fa_s, fa_r)
            s_send(_BA, 0, s, bwd_tgt, ba_s, ba_r)

        for r in range(2, NO):
            for s in range(SUB):
                s_recv(_FO, r - 1, s, fwd_tgt, fo_s, fo_r)
                s_send(_FO, r, s, fwd_tgt, fo_s, fo_r)
            for s in range(SUB):
                s_recv(_BO, r - 1, s, bwd_tgt, bo_s, bo_r)
                s_send(_BO, r, s, bwd_tgt, bo_s, bo_r)
            for s in range(SUB):
                s_recv(_FA, r - 2, s, fwd_tgt, fa_s, fa_r)
                s_send(_FA, r - 1, s, fwd_tgt, fa_s, fa_r)
            for s in range(SUB):
                s_recv(_BA, r - 2, s, bwd_tgt, ba_s, ba_r)
                s_send(_BA, r - 1, s, bwd_tgt, ba_s, ba_r)

        for s in range(SUB):
            s_recv(_FO, NO - 1, s, fwd_tgt, fo_s, fo_r)
        for s in range(SUB):
            s_recv(_BO, NO - 1, s, bwd_tgt, bo_s, bo_r)
        for s in range(SUB):
            s_recv(_FA, NA - 1, s, fwd_tgt, fa_s, fa_r)
        for s in range(SUB):
            s_recv(_BA, NA - 1, s, bwd_tgt, ba_s, ba_r)

        wbf = w_ref[...].astype(jnp.bfloat16)
        y = jnp.dot(gx_ref[...], wbf, preferred_element_type=jnp.float32)
        c = 0.7978845608028654
        out_ref[...] = 0.5 * y * (1.0 + jnp.tanh(c * (y + 0.044715 * y * y * y)))

        for d in sends:
            d.wait_send()

    return pl.pallas_call(
        body,
        out_shape=jax.ShapeDtypeStruct((m_glob, n_per), jnp.float32),
        in_specs=[
            pl.BlockSpec(memory_space=pltpu.VMEM),
            pl.BlockSpec(memory_space=pltpu.VMEM),
            pl.BlockSpec(memory_space=pltpu.SMEM),
        ],
        out_specs=pl.BlockSpec(memory_space=pltpu.VMEM),
        scratch_shapes=[
            pltpu.VMEM((m_glob, k), jnp.bfloat16),
            pltpu.SemaphoreType.DMA((SUB,)),
            pltpu.SemaphoreType.DMA((SUB,)),
            pltpu.SemaphoreType.DMA((NO, SUB)),
            pltpu.SemaphoreType.DMA((NO, SUB)),
            pltpu.SemaphoreType.DMA((NA, SUB)),
            pltpu.SemaphoreType.DMA((NA, SUB)),
            pltpu.SemaphoreType.DMA((NO, SUB)),
            pltpu.SemaphoreType.DMA((NO, SUB)),
            pltpu.SemaphoreType.DMA((NA, SUB)),
            pltpu.SemaphoreType.DMA((NA, SUB)),
        ],
        compiler_params=pltpu.CompilerParams(collective_id=0),
    )(x, w_mat, meta)
